# Initial kernel scaffold; baseline (speedup 1.0000x reference)
import jax
import jax.numpy as jnp
from jax.experimental import pallas as pl


def kernel(coords, bond_rest_length, bond_half_force_constant, angle_force_constant, angle_c0, angle_c1, angle_c2, torsion_half_force_constant, torsion_cos_term, vdw_minimum, vdw_well_depth, bond_index, angle_index, torsion_index, torsion_order, nonbond_index):
    raise NotImplementedError("write your pallas kernel here")


# speedup vs baseline: 31.6921x; 31.6921x over previous
import functools

import jax
import jax.numpy as jnp
from jax import lax
from jax.experimental import pallas as pl
from jax.experimental.pallas import tpu as pltpu
from jax.experimental.pallas import tpu_sc as plsc

NW = 32
C = 128
EPS = 1e-8


def _rsqrt(s):
    b = lax.bitcast_convert_type(s, jnp.int32)
    y = lax.bitcast_convert_type(jnp.int32(0x5F3759DF) - (b >> 1), jnp.float32)
    h = 0.5 * s
    for _ in range(3):
        y = y * (1.5 - h * y * y)
    return y


def _load_xyz(rows_ref, g):
    ridx = lax.iota(jnp.int32, 16) + g * 16
    x = plsc.load_gather(rows_ref, [ridx, jnp.full((16,), 0, jnp.int32)])
    y = plsc.load_gather(rows_ref, [ridx, jnp.full((16,), 1, jnp.int32)])
    z = plsc.load_gather(rows_ref, [ridx, jnp.full((16,), 2, jnp.int32)])
    return x, y, z


def _term_loop(nc, wid, coords, idx_hbm, par_hbm, idx_v, rows_v, par_v,
               sem_idx, sem_data, acc, compute_group):
    S = len(idx_hbm)
    P = len(par_hbm)

    def issue_idx(c, b):
        for s in range(S):
            pltpu.async_copy(idx_hbm[s].at[wid, c], idx_v[s][b], sem_idx[b])

    def wait_idx(c, b):
        for s in range(S):
            pltpu.make_async_copy(idx_hbm[s].at[wid, c], idx_v[s][b],
                                  sem_idx[b]).wait()

    def issue_data(c, b):
        for s in range(S):
            pltpu.async_copy(coords.at[idx_v[s][b]], rows_v[s][b], sem_data[b])
        for p in range(P):
            pltpu.async_copy(par_hbm[p].at[wid, c], par_v[p][b], sem_data[b])

    def wait_data(c, b):
        for s in range(S):
            pltpu.make_async_copy(coords.at[idx_v[s][b]], rows_v[s][b],
                                  sem_data[b]).wait()
        for p in range(P):
            pltpu.make_async_copy(par_hbm[p].at[wid, c], par_v[p][b],
                                  sem_data[b]).wait()

    def compute(b, acc):
        for g in range(C // 16):
            pars = [par_v[p][b][pl.ds(g * 16, 16)] for p in range(P)]
            acc = acc + compute_group(g, b, pars)
        return acc

    for s in range(S):
        pltpu.sync_copy(idx_hbm[s].at[wid, 0], idx_v[s][0])
    issue_idx(1, 1)
    issue_data(0, 0)

    def body(j, acc):
        c = 2 * j
        for (cc, b) in ((c, 0), (c + 1, 1)):
            wait_idx(cc + 1, 1 - b)
            issue_data(cc + 1, 1 - b)
            wait_data(cc, b)
            issue_idx(cc + 2, b)
            acc = compute(b, acc)
        return acc

    acc = lax.fori_loop(0, nc // 2 - 1, body, acc)

    wait_idx(nc - 1, 1)
    issue_data(nc - 1, 1)
    wait_data(nc - 2, 0)
    acc = compute(0, acc)
    wait_data(nc - 1, 1)
    acc = compute(1, acc)
    return acc


def _uff_sc(nc_b, nc_a, nc_t, nc_n):
    mesh = plsc.VectorSubcoreMesh(core_axis_name="c", subcore_axis_name="s",
                                  num_cores=2, num_subcores=16)

    @functools.partial(
        pl.kernel,
        out_type=jax.ShapeDtypeStruct((NW, 16), jnp.float32),
        mesh=mesh,
        compiler_params=pltpu.CompilerParams(
            needs_layout_passes=False, use_tc_tiling_on_sc=False),
        scratch_types=[
            [[pltpu.VMEM((C,), jnp.int32) for _ in range(2)]
             for _ in range(4)],
            [[pltpu.VMEM((C, 3), jnp.float32) for _ in range(2)]
             for _ in range(4)],
            [[pltpu.VMEM((C,), jnp.float32) for _ in range(2)]
             for _ in range(4)],
            [pltpu.VMEM((C,), jnp.int32) for _ in range(2)],
            pltpu.VMEM((16,), jnp.float32),
            [pltpu.SemaphoreType.DMA for _ in range(2)],
            [pltpu.SemaphoreType.DMA for _ in range(2)],
        ],
    )
    def k(coords, b_i0, b_i1, b_r0, b_k,
          a_i0, a_i1, a_i2, a_k, a_c0, a_c1, a_c2,
          t_i0, t_i1, t_i2, t_i3, t_k, t_ct, t_ord,
          n_i0, n_i1, n_rm, n_dd,
          out, idx_v, rows_v, par_v, ord_v, acc_v, sem_idx, sem_data):
        wid = lax.axis_index("s") * 2 + lax.axis_index("c")
        acc = jnp.zeros((16,), jnp.float32)

        def bond_group(g, b, pars):
            r0, hk = pars
            xa, ya, za = _load_xyz(rows_v[0][b], g)
            xb, yb, zb = _load_xyz(rows_v[1][b], g)
            dx, dy, dz = xa - xb, ya - yb, za - zb
            s = dx * dx + dy * dy + dz * dz + EPS
            r = s * _rsqrt(s)
            dr = r - r0
            return hk * dr * dr

        acc = _term_loop(nc_b, wid, coords, [b_i0, b_i1], [b_r0, b_k],
                         idx_v, rows_v, par_v, sem_idx, sem_data, acc,
                         bond_group)

        def angle_group(g, b, pars):
            ak, c0, c1, c2 = pars
            xi, yi, zi = _load_xyz(rows_v[0][b], g)
            xj, yj, zj = _load_xyz(rows_v[1][b], g)
            xk, yk, zk = _load_xyz(rows_v[2][b], g)
            v1x, v1y, v1z = xi - xj, yi - yj, zi - zj
            v2x, v2y, v2z = xk - xj, yk - yj, zk - zj
            q1 = v1x * v1x + v1y * v1y + v1z * v1z + EPS
            q2 = v2x * v2x + v2y * v2y + v2z * v2z + EPS
            dt = v1x * v2x + v1y * v2y + v1z * v2z
            cos = jnp.clip(dt * _rsqrt(q1 * q2), -0.9999, 0.9999)
            return ak * (c0 + c1 * cos + c2 * (2.0 * cos * cos - 1.0))

        acc = _term_loop(nc_a, wid, coords, [a_i0, a_i1, a_i2],
                         [a_k, a_c0, a_c1, a_c2],
                         idx_v, rows_v, par_v, sem_idx, sem_data, acc,
                         angle_group)

        def torsion_group(g, b, pars):
            hk, ct = pars
            order = ord_v[b][pl.ds(g * 16, 16)]
            x0, y0, z0 = _load_xyz(rows_v[0][b], g)
            x1, y1, z1 = _load_xyz(rows_v[1][b], g)
            x2, y2, z2 = _load_xyz(rows_v[2][b], g)
            x3, y3, z3 = _load_xyz(rows_v[3][b], g)
            b1x, b1y, b1z = x1 - x0, y1 - y0, z1 - z0
            b2x, b2y, b2z = x2 - x1, y2 - y1, z2 - z1
            b3x, b3y, b3z = x3 - x2, y3 - y2, z3 - z2
            c1x = b1y * b2z - b1z * b2y
            c1y = b1z * b2x - b1x * b2z
            c1z = b1x * b2y - b1y * b2x
            c2x = b2y * b3z - b2z * b3y
            c2y = b2z * b3x - b2x * b3z
            c2z = b2x * b3y - b2y * b3x
            m1 = c1x * c1x + c1y * c1y + c1z * c1z + EPS
            m2 = c2x * c2x + c2y * c2y + c2z * c2z + EPS
            dt = c1x * c2x + c1y * c2y + c1z * c2z
            cos = jnp.clip(dt * _rsqrt(m1 * m2), -0.9999, 0.9999)
            cos2 = 2.0 * cos * cos - 1.0
            cos3 = cos * (4.0 * cos * cos - 3.0)
            cosn = jnp.where(order == 1, cos,
                             jnp.where(order == 2, cos2, cos3))
            return hk * (1.0 - ct * cosn)

        acc = _term_loop(nc_t, wid, coords, [t_i0, t_i1, t_i2, t_i3],
                         [t_k, t_ct, t_ord],
                         idx_v, rows_v, [par_v[0], par_v[1], ord_v],
                         sem_idx, sem_data, acc,
                         lambda g, b, pars: torsion_group(g, b, pars[:2]))

        def vdw_group(g, b, pars):
            rm, dd = pars
            xa, ya, za = _load_xyz(rows_v[0][b], g)
            xb, yb, zb = _load_xyz(rows_v[1][b], g)
            dx, dy, dz = xa - xb, ya - yb, za - zb
            r2 = jnp.maximum(dx * dx + dy * dy + dz * dz + EPS, 0.25)
            t = (rm * rm) / r2
            x6 = t * t * t
            return dd * x6 * (x6 - 2.0)

        acc = _term_loop(nc_n, wid, coords, [n_i0, n_i1], [n_rm, n_dd],
                         idx_v, rows_v, par_v, sem_idx, sem_data, acc,
                         vdw_group)

        acc_v[...] = acc
        pltpu.sync_copy(acc_v, out.at[wid])

    return k


def _prep(arr, total, pad_val=0):
    t = arr.shape[0]
    if total > t:
        arr = jnp.concatenate(
            [arr, jnp.full((total - t,), pad_val, arr.dtype)])
    return arr.reshape(NW, total // (NW * C), C)


def kernel(coords, bond_rest_length, bond_half_force_constant,
           angle_force_constant, angle_c0, angle_c1, angle_c2,
           torsion_half_force_constant, torsion_cos_term, vdw_minimum,
           vdw_well_depth, bond_index, angle_index, torsion_index,
           torsion_order, nonbond_index):
    unit = NW * C * 2

    def up(t):
        return max(2, (t + unit - 1) // unit) * unit

    NB, NA = bond_index.shape[0], angle_index.shape[0]
    NT, NP = torsion_index.shape[0], nonbond_index.shape[0]
    NBp, NAp, NTp, NPp = up(NB), up(NA), up(NT), up(NP)

    args = [coords]
    args += [_prep(bond_index[:, s], NBp) for s in range(2)]
    args += [_prep(p, NBp) for p in (bond_rest_length, bond_half_force_constant)]
    args += [_prep(angle_index[:, s], NAp) for s in range(3)]
    args += [_prep(p, NAp) for p in (angle_force_constant, angle_c0, angle_c1, angle_c2)]
    args += [_prep(torsion_index[:, s], NTp) for s in range(4)]
    args += [_prep(p, NTp) for p in (torsion_half_force_constant, torsion_cos_term)]
    args += [_prep(torsion_order, NTp)]
    args += [_prep(nonbond_index[:, s], NPp) for s in range(2)]
    args += [_prep(p, NPp) for p in (vdw_minimum, vdw_well_depth)]

    k = _uff_sc(NBp // (NW * C), NAp // (NW * C), NTp // (NW * C),
                NPp // (NW * C))
    partials = k(*args)
    return jnp.sum(partials)
